# baseline (speedup 1.0000x reference)
import jax
import jax.numpy as jnp
from jax.experimental import pallas as pl

N = 10000
B = 64


def _readout_body(pooled_ref, Wl1a_ref, Wl1b_ref, Wl2a_ref, Wl2b_ref, Wout_ref, out_ref):
    z = jnp.maximum(pooled_ref[...] @ Wl1a_ref[...], 0.0)
    z = jnp.maximum(z @ Wl1b_ref[...], 0.0)
    z = jnp.maximum(z @ Wl2a_ref[...], 0.0)
    z = z @ Wl2b_ref[...]
    out_ref[...] = z @ Wout_ref[...]


def kernel(x, edge_index, edge_attr, batch, W_node, W_edge, Wcm1, Wcm2, Wcm3,
           Wn1, Wn2, We1, We2, Wc1, Wc2, eps_arr, Wl1a, Wl1b, Wl2a, Wl2b, Wout):
    src = edge_index[0]
    dst = edge_index[1]
    h = x @ W_node
    e = edge_attr @ W_edge
    g1 = jax.ops.segment_sum(h[src], dst, num_segments=N)
    g2 = jax.ops.segment_sum(g1[src], dst, num_segments=N)
    cyc = jnp.concatenate([g1, g2], axis=-1)
    cyc = jax.nn.relu(cyc @ Wcm1)
    cyc = jax.nn.relu(cyc @ Wcm2)
    cyc = cyc @ Wcm3
    L = Wn1.shape[0]
    for i in range(L):
        agg = jax.ops.segment_sum(h[src] + e, dst, num_segments=N)
        h_new = (1.0 + eps_arr[i]) * h + agg
        h_new = jax.nn.relu(h_new @ Wn1[i])
        h = jax.nn.relu(h_new @ Wn2[i])
        e_new = e + h[src] + h[dst]
        e_new = jax.nn.relu(e_new @ We1[i])
        e = jax.nn.relu(e_new @ We2[i])
        c_new = cyc + h
        c_new = jax.nn.relu(c_new @ Wc1[i])
        cyc = jax.nn.relu(c_new @ Wc2[i])
    edge2node = jax.ops.segment_sum(e, dst, num_segments=N)
    feats = jnp.concatenate([h, edge2node, cyc], axis=-1)
    pooled = jax.ops.segment_sum(feats, batch, num_segments=B)
    out = pl.pallas_call(
        _readout_body,
        out_shape=jax.ShapeDtypeStruct((B, 1), jnp.float32),
    )(pooled, Wl1a, Wl1b, Wl2a, Wl2b, Wout)
    return out


# speedup vs baseline: 1.0847x; 1.0847x over previous
import functools

import jax
import jax.numpy as jnp
from jax import lax
from jax.experimental import pallas as pl
from jax.experimental.pallas import tpu as pltpu
from jax.experimental.pallas import tpu_sc as plsc

N = 10000
E = 320000
H = 128
B = 64

NC = 2
NS = 16
NW = NC * NS
CHUNK = 128
NCHUNKS = E // CHUNK
NPAD = 10240
ROWS_PER_TILE = NPAD // NW
ROWS_OWN = NPAD // NW
BUF_ROWS = ROWS_OWN + 8

_MESH = plsc.VectorSubcoreMesh(core_axis_name="c", subcore_axis_name="s")


def _extract_pair(vec48, t):
    w = vec48[pl.ds(t, 16)]
    return w[0], w[1]


def _bcast_elem(w16, k):
    idx = jnp.full((16, 1), 0, jnp.int32) + k
    dn = lax.GatherDimensionNumbers(offset_dims=(), collapsed_slice_dims=(0,),
                                    start_index_map=(0,))
    return lax.gather(w16, idx, dn, (1,), mode=lax.GatherScatterMode.PROMISE_IN_BOUNDS)


def _segsum_body(with_gather, with_e, *refs):
    idx = 0
    table_hbm = None
    e_hbm = None
    if with_gather:
        table_hbm = refs[idx]; idx += 1
    if with_e:
        e_hbm = refs[idx]; idx += 1
    src_hbm = refs[idx]; idx += 1
    dst_hbm = refs[idx]; idx += 1
    ep_hbm = refs[idx]; idx += 1
    zeros_hbm = refs[idx]; idx += 1
    out_hbm = refs[idx]; idx += 1
    (sidx_v, didx_v, trows_v, erows_v, ep_v, acc_v, sem) = refs[idx:]

    cid = lax.axis_index("c")
    sid = lax.axis_index("s")
    tid = cid * NS + sid
    rlo = tid * ROWS_OWN
    rhi = rlo + ROWS_OWN

    pltpu.sync_copy(ep_hbm, ep_v)
    estart, eend = _extract_pair(ep_v, tid)
    cstart = estart // CHUNK
    cend = (eend + (CHUNK - 1)) // CHUNK

    pltpu.sync_copy(zeros_hbm, acc_v)

    def do_chunk(j, carry):
        base = (cstart + j) * CHUNK
        pltpu.sync_copy(dst_hbm.at[pl.ds(base, CHUNK)], didx_v)
        if with_gather:
            pltpu.sync_copy(src_hbm.at[pl.ds(base, CHUNK)], sidx_v)
            pltpu.async_copy(table_hbm.at[sidx_v], trows_v, sem).wait()
        if with_e:
            pltpu.sync_copy(e_hbm.at[pl.ds(base, CHUNK), :], erows_v)

        def per_edge(i, c2):
            lanes = lax.iota(jnp.int32, 16)
            zero16 = jnp.full((16,), 0, jnp.int32)
            w16 = didx_v[pl.ds((i // 16) * 16, 16)]
            d_b = _bcast_elem(w16, i % 16)
            inr = (d_b >= rlo) & (d_b < rhi)
            rowv = jnp.where(inr, d_b - rlo, jnp.int32(ROWS_OWN))
            for g in range(8):
                colv = lanes + (16 * g)
                if with_gather and with_e:
                    u = (trows_v[i, pl.ds(16 * g, 16)]
                         + erows_v[i, pl.ds(16 * g, 16)])
                elif with_gather:
                    u = trows_v[i, pl.ds(16 * g, 16)]
                else:
                    u = erows_v[i, pl.ds(16 * g, 16)]
                plsc.addupdate_scatter(acc_v, [rowv * H + colv], u)
            return c2

        lax.fori_loop(0, CHUNK, per_edge, 0)
        return carry

    lax.fori_loop(0, cend - cstart, do_chunk, 0)

    pltpu.sync_copy(acc_v.at[pl.ds(0, ROWS_OWN * H)],
                    out_hbm.at[pl.ds(rlo * H, ROWS_OWN * H)])


def _make_segsum(with_gather, with_e):
    scratch = [
        pltpu.VMEM((CHUNK,), jnp.int32),
        pltpu.VMEM((CHUNK,), jnp.int32),
        pltpu.VMEM((CHUNK, H), jnp.float32),
        pltpu.VMEM((CHUNK, H), jnp.float32),
        pltpu.VMEM((48,), jnp.int32),
        pltpu.VMEM((BUF_ROWS * H,), jnp.float32),
        pltpu.SemaphoreType.DMA,
    ]
    return functools.partial(
        pl.kernel,
        functools.partial(_segsum_body, with_gather, with_e),
        mesh=_MESH,
        out_type=jax.ShapeDtypeStruct((NPAD * H,), jnp.float32),
        scratch_types=scratch,
        compiler_params=pltpu.CompilerParams(needs_layout_passes=False),
    )()


def _zeros_buf():
    return jnp.zeros((BUF_ROWS * H,), jnp.float32)


def _segsum_gather(table, ssrc, sdst, ep):
    p = _make_segsum(True, False)(table, ssrc, sdst, ep, _zeros_buf())
    return p.reshape(NPAD, H)[:N]


def _segsum_gather_plus_e(table, e, ssrc, sdst, ep):
    p = _make_segsum(True, True)(table, e, ssrc, sdst, ep, _zeros_buf())
    return p.reshape(NPAD, H)[:N]


def _segsum_e(e, sdst, ep):
    p = _make_segsum(False, True)(e, sdst, sdst, ep, _zeros_buf())
    return p.reshape(NPAD, H)[:N]


E_PER_TILE = E // NW
NFULL = E_PER_TILE // CHUNK
TAIL = E_PER_TILE - NFULL * CHUNK


def _edge_update_body(h_hbm, e_hbm, src_hbm, dst_hbm, out_hbm,
                      sidx_v, didx_v, hs_v, hd_v, er_v,
                      sidx_t, didx_t, hs_t, hd_t, er_t, sem):
    cid = lax.axis_index("c")
    sid = lax.axis_index("s")
    tile_base = cid * (E // NC) + sid * E_PER_TILE

    def do_chunk(base, k, sidx, didx, hs, hd, er):
        pltpu.sync_copy(src_hbm.at[pl.ds(base, k)], sidx)
        pltpu.sync_copy(dst_hbm.at[pl.ds(base, k)], didx)
        pltpu.async_copy(h_hbm.at[sidx], hs, sem).wait()
        pltpu.async_copy(h_hbm.at[didx], hd, sem).wait()
        pltpu.sync_copy(e_hbm.at[pl.ds(base, k), :], er)

        def add_row(i, carry):
            r = i // 8
            c = (i % 8) * 16
            er[r, pl.ds(c, 16)] = ((er[r, pl.ds(c, 16)]
                                    + hs[r, pl.ds(c, 16)])
                                   + hd[r, pl.ds(c, 16)])
            return carry

        lax.fori_loop(0, k * 8, add_row, 0)
        pltpu.sync_copy(er, out_hbm.at[pl.ds(base, k), :])

    def loop_body(j, carry):
        do_chunk(tile_base + j * CHUNK, CHUNK, sidx_v, didx_v, hs_v, hd_v, er_v)
        return carry

    lax.fori_loop(0, NFULL, loop_body, 0)
    do_chunk(tile_base + NFULL * CHUNK, TAIL, sidx_t, didx_t, hs_t, hd_t, er_t)


def _edge_update(h, e, ssrc, sdst):
    scratch = [
        pltpu.VMEM((CHUNK,), jnp.int32),
        pltpu.VMEM((CHUNK,), jnp.int32),
        pltpu.VMEM((CHUNK, H), jnp.float32),
        pltpu.VMEM((CHUNK, H), jnp.float32),
        pltpu.VMEM((CHUNK, H), jnp.float32),
        pltpu.VMEM((TAIL,), jnp.int32),
        pltpu.VMEM((TAIL,), jnp.int32),
        pltpu.VMEM((TAIL, H), jnp.float32),
        pltpu.VMEM((TAIL, H), jnp.float32),
        pltpu.VMEM((TAIL, H), jnp.float32),
        pltpu.SemaphoreType.DMA,
    ]
    f = functools.partial(
        pl.kernel,
        _edge_update_body,
        mesh=_MESH,
        out_type=jax.ShapeDtypeStruct((E, H), jnp.float32),
        scratch_types=scratch,
    )()
    return f(h, e, ssrc, sdst)


def _permute_rows_body(ea_hbm, perm_hbm, out_hbm,
                       pidx_v, rows_v, pidx_t, rows_t, sem):
    cid = lax.axis_index("c")
    sid = lax.axis_index("s")
    tile_base = cid * (E // NC) + sid * E_PER_TILE

    def do_chunk(base, k, pidx, rows):
        pltpu.sync_copy(perm_hbm.at[pl.ds(base, k)], pidx)
        pltpu.async_copy(ea_hbm.at[pidx], rows, sem).wait()
        pltpu.sync_copy(rows, out_hbm.at[pl.ds(base, k), :])

    def loop_body(j, carry):
        do_chunk(tile_base + j * CHUNK, CHUNK, pidx_v, rows_v)
        return carry

    lax.fori_loop(0, NFULL, loop_body, 0)
    do_chunk(tile_base + NFULL * CHUNK, TAIL, pidx_t, rows_t)


def _permute_rows(table, perm):
    scratch = [
        pltpu.VMEM((CHUNK,), jnp.int32),
        pltpu.VMEM((CHUNK, H), jnp.float32),
        pltpu.VMEM((TAIL,), jnp.int32),
        pltpu.VMEM((TAIL, H), jnp.float32),
        pltpu.SemaphoreType.DMA,
    ]
    f = functools.partial(
        pl.kernel,
        _permute_rows_body,
        mesh=_MESH,
        out_type=jax.ShapeDtypeStruct((E, H), jnp.float32),
        scratch_types=scratch,
    )()
    return f(table, perm)


CHUNK_P = 80
NCHUNKS_P = N // CHUNK_P
GR_OWN = B // NW
PBUF_ROWS = 16


def _pool_body(f0_hbm, f1_hbm, f2_hbm, b_hbm, gp_hbm, zeros_hbm,
               o0_hbm, o1_hbm, o2_hbm,
               bidx_v, r0_v, r1_v, r2_v, gp_v, a0_v, a1_v, a2_v, sem):
    cid = lax.axis_index("c")
    sid = lax.axis_index("s")
    tid = cid * NS + sid
    glo = tid * GR_OWN
    ghi = glo + GR_OWN

    pltpu.sync_copy(gp_hbm, gp_v)
    nstart, nend = _extract_pair(gp_v, tid)
    cstart = nstart // CHUNK_P
    cend = (nend + (CHUNK_P - 1)) // CHUNK_P

    pltpu.sync_copy(zeros_hbm, a0_v)
    pltpu.sync_copy(zeros_hbm, a1_v)
    pltpu.sync_copy(zeros_hbm, a2_v)

    def do_chunk(j, carry):
        base = (cstart + j) * CHUNK_P
        pltpu.sync_copy(b_hbm.at[pl.ds(base, CHUNK_P)], bidx_v.at[pl.ds(0, CHUNK_P)])
        pltpu.sync_copy(f0_hbm.at[pl.ds(base, CHUNK_P), :], r0_v)
        pltpu.sync_copy(f1_hbm.at[pl.ds(base, CHUNK_P), :], r1_v)
        pltpu.sync_copy(f2_hbm.at[pl.ds(base, CHUNK_P), :], r2_v)

        def per_node(i, c2):
            lanes = lax.iota(jnp.int32, 16)
            zero16 = jnp.full((16,), 0, jnp.int32)
            w16 = bidx_v[pl.ds((i // 16) * 16, 16)]
            b_b = _bcast_elem(w16, i % 16)
            inr = (b_b >= glo) & (b_b < ghi)
            rowv = jnp.where(inr, b_b - glo, jnp.int32(8))
            for g in range(8):
                colv = lanes + (16 * g)
                fidx = rowv * H + colv
                plsc.addupdate_scatter(a0_v, [fidx], r0_v[i, pl.ds(16 * g, 16)])
                plsc.addupdate_scatter(a1_v, [fidx], r1_v[i, pl.ds(16 * g, 16)])
                plsc.addupdate_scatter(a2_v, [fidx], r2_v[i, pl.ds(16 * g, 16)])
            return c2

        lax.fori_loop(0, CHUNK_P, per_node, 0)
        return carry

    lax.fori_loop(0, cend - cstart, do_chunk, 0)

    pltpu.sync_copy(a0_v.at[pl.ds(0, 8 * H)], o0_hbm.at[pl.ds(tid * 8 * H, 8 * H)])
    pltpu.sync_copy(a1_v.at[pl.ds(0, 8 * H)], o1_hbm.at[pl.ds(tid * 8 * H, 8 * H)])
    pltpu.sync_copy(a2_v.at[pl.ds(0, 8 * H)], o2_hbm.at[pl.ds(tid * 8 * H, 8 * H)])


def _pool(h, e2n, cyc, batch, gp):
    scratch = [
        pltpu.VMEM((128,), jnp.int32),
        pltpu.VMEM((CHUNK_P, H), jnp.float32),
        pltpu.VMEM((CHUNK_P, H), jnp.float32),
        pltpu.VMEM((CHUNK_P, H), jnp.float32),
        pltpu.VMEM((48,), jnp.int32),
        pltpu.VMEM((PBUF_ROWS * H,), jnp.float32),
        pltpu.VMEM((PBUF_ROWS * H,), jnp.float32),
        pltpu.VMEM((PBUF_ROWS * H,), jnp.float32),
        pltpu.SemaphoreType.DMA,
    ]
    out_t = [jax.ShapeDtypeStruct((NW * 8 * H,), jnp.float32)] * 3
    f = functools.partial(
        pl.kernel,
        _pool_body,
        mesh=_MESH,
        out_type=out_t,
        scratch_types=scratch,
        compiler_params=pltpu.CompilerParams(needs_layout_passes=False),
    )()
    zeros = jnp.zeros((PBUF_ROWS * H,), jnp.float32)
    o0, o1, o2 = f(h, e2n, cyc, batch, gp, zeros)
    p0 = o0.reshape(NW, 8, H)[:, :GR_OWN].reshape(B, H)
    p1 = o1.reshape(NW, 8, H)[:, :GR_OWN].reshape(B, H)
    p2 = o2.reshape(NW, 8, H)[:, :GR_OWN].reshape(B, H)
    return jnp.concatenate([p0, p1, p2], axis=-1)


def kernel(x, edge_index, edge_attr, batch, W_node, W_edge, Wcm1, Wcm2, Wcm3,
           Wn1, Wn2, We1, We2, Wc1, Wc2, eps_arr, Wl1a, Wl1b, Wl2a, Wl2b, Wout):
    src = edge_index[0].astype(jnp.int32)
    dst = edge_index[1].astype(jnp.int32)
    batch = batch.astype(jnp.int32)

    perm = jnp.argsort(dst, stable=True).astype(jnp.int32)
    sdst = dst[perm]
    ssrc = src[perm]
    bounds = jnp.arange(0, NW + 1, dtype=jnp.int32) * ROWS_OWN
    ep = jnp.searchsorted(sdst, bounds, side="left").astype(jnp.int32)
    ep = jnp.pad(ep, (0, 48 - (NW + 1)))
    gbounds = jnp.arange(0, NW + 1, dtype=jnp.int32) * GR_OWN
    gp = jnp.searchsorted(batch, gbounds, side="left").astype(jnp.int32)
    gp = jnp.pad(gp, (0, 48 - (NW + 1)))

    h = x @ W_node
    e = _permute_rows(edge_attr @ W_edge, perm)

    g1 = _segsum_gather(h, ssrc, sdst, ep)
    g2 = _segsum_gather(g1, ssrc, sdst, ep)
    cyc = jnp.concatenate([g1, g2], axis=-1)
    cyc = jax.nn.relu(cyc @ Wcm1)
    cyc = jax.nn.relu(cyc @ Wcm2)
    cyc = cyc @ Wcm3
    L = Wn1.shape[0]
    for i in range(L):
        agg = _segsum_gather_plus_e(h, e, ssrc, sdst, ep)
        h_new = (1.0 + eps_arr[i]) * h + agg
        h_new = jax.nn.relu(h_new @ Wn1[i])
        h = jax.nn.relu(h_new @ Wn2[i])
        e_new = _edge_update(h, e, ssrc, sdst)
        e_new = jax.nn.relu(e_new @ We1[i])
        e = jax.nn.relu(e_new @ We2[i])
        c_new = cyc + h
        c_new = jax.nn.relu(c_new @ Wc1[i])
        cyc = jax.nn.relu(c_new @ Wc2[i])
    edge2node = _segsum_e(e, sdst, ep)
    pooled = _pool(h, edge2node, cyc, batch, gp)
    z = jax.nn.relu(pooled @ Wl1a)
    z = jax.nn.relu(z @ Wl1b)
    z = jax.nn.relu(z @ Wl2a)
    z = z @ Wl2b
    out = z @ Wout
    return out


# speedup vs baseline: 1.4177x; 1.3070x over previous
import functools

import jax
import jax.numpy as jnp
from jax import lax
from jax.experimental import pallas as pl
from jax.experimental.pallas import tpu as pltpu
from jax.experimental.pallas import tpu_sc as plsc

N = 10000
E = 320000
H = 128
B = 64

NC = 2
NS = 16
NW = NC * NS
CHUNK = 128
NCHUNKS = E // CHUNK
NPAD = 10240
ROWS_PER_TILE = NPAD // NW
ROWS_OWN = NPAD // NW
BUF_ROWS = ROWS_OWN + 8

_MESH = plsc.VectorSubcoreMesh(core_axis_name="c", subcore_axis_name="s")


def _extract_pair(vec48, t):
    w = vec48[pl.ds(t, 16)]
    return w[0], w[1]


def _bcast_elem(w16, k):
    idx = jnp.full((16, 1), 0, jnp.int32) + k
    dn = lax.GatherDimensionNumbers(offset_dims=(), collapsed_slice_dims=(0,),
                                    start_index_map=(0,))
    return lax.gather(w16, idx, dn, (1,), mode=lax.GatherScatterMode.PROMISE_IN_BOUNDS)


def _segsum_body(with_gather, with_e, *refs):
    idx = 0
    table_hbm = None
    e_hbm = None
    if with_gather:
        table_hbm = refs[idx]; idx += 1
    if with_e:
        e_hbm = refs[idx]; idx += 1
    src_hbm = refs[idx]; idx += 1
    dst_hbm = refs[idx]; idx += 1
    ep_hbm = refs[idx]; idx += 1
    zeros_hbm = refs[idx]; idx += 1
    out_hbm = refs[idx]; idx += 1
    (sidx_v, didx_v, trows_v, erows_v, ep_v, acc_v, sem) = refs[idx:]

    cid = lax.axis_index("c")
    sid = lax.axis_index("s")
    tid = cid * NS + sid
    rlo = tid * ROWS_OWN
    rhi = rlo + ROWS_OWN

    pltpu.sync_copy(ep_hbm, ep_v)
    estart, eend = _extract_pair(ep_v, tid)
    cstart = estart // CHUNK
    cend = (eend + (CHUNK - 1)) // CHUNK

    pltpu.sync_copy(zeros_hbm, acc_v)

    def do_chunk(j, carry):
        base = (cstart + j) * CHUNK
        copies = []
        copies.append(pltpu.async_copy(dst_hbm.at[pl.ds(base, CHUNK)], didx_v, sem))
        if with_gather:
            pltpu.sync_copy(src_hbm.at[pl.ds(base, CHUNK)], sidx_v)
            copies.append(pltpu.async_copy(table_hbm.at[sidx_v], trows_v, sem))
        if with_e:
            copies.append(pltpu.async_copy(e_hbm.at[pl.ds(base, CHUNK), :], erows_v, sem))
        for c in copies:
            c.wait()

        def per_group(gi, c2):
            lanes = lax.iota(jnp.int32, 16)
            w16 = didx_v[pl.ds(gi * 16, 16)]
            inr = (w16 >= rlo) & (w16 < rhi)
            rowflat = jnp.where(inr, (w16 - rlo) * H, jnp.int32(ROWS_OWN * H))
            for jj in range(16):
                rb = _bcast_elem(rowflat, jj)
                i = gi * 16 + jj
                us = []
                for g in range(8):
                    if with_gather and with_e:
                        u = (trows_v[i, pl.ds(16 * g, 16)]
                             + erows_v[i, pl.ds(16 * g, 16)])
                    elif with_gather:
                        u = trows_v[i, pl.ds(16 * g, 16)]
                    else:
                        u = erows_v[i, pl.ds(16 * g, 16)]
                    us.append(u)
                idxs = [rb + (lanes + 16 * g) for g in range(8)]
                for g in range(8):
                    plsc.addupdate_scatter(acc_v, [idxs[g]], us[g])
            return c2

        lax.fori_loop(0, 8, per_group, 0)
        return carry

    lax.fori_loop(0, cend - cstart, do_chunk, 0)

    pltpu.sync_copy(acc_v.at[pl.ds(0, ROWS_OWN * H)],
                    out_hbm.at[pl.ds(rlo * H, ROWS_OWN * H)])


def _make_segsum(with_gather, with_e):
    scratch = [
        pltpu.VMEM((CHUNK,), jnp.int32),
        pltpu.VMEM((CHUNK,), jnp.int32),
        pltpu.VMEM((CHUNK, H), jnp.float32),
        pltpu.VMEM((CHUNK, H), jnp.float32),
        pltpu.VMEM((48,), jnp.int32),
        pltpu.VMEM((BUF_ROWS * H,), jnp.float32),
        pltpu.SemaphoreType.DMA,
    ]
    return functools.partial(
        pl.kernel,
        functools.partial(_segsum_body, with_gather, with_e),
        mesh=_MESH,
        out_type=jax.ShapeDtypeStruct((NPAD * H,), jnp.float32),
        scratch_types=scratch,
        compiler_params=pltpu.CompilerParams(needs_layout_passes=False),
    )()


def _zeros_buf():
    return jnp.zeros((BUF_ROWS * H,), jnp.float32)


def _segsum_gather(table, ssrc, sdst, ep):
    p = _make_segsum(True, False)(table, ssrc, sdst, ep, _zeros_buf())
    return p.reshape(NPAD, H)[:N]


def _segsum_gather_plus_e(table, e, ssrc, sdst, ep):
    p = _make_segsum(True, True)(table, e, ssrc, sdst, ep, _zeros_buf())
    return p.reshape(NPAD, H)[:N]


def _segsum_e(e, sdst, ep):
    p = _make_segsum(False, True)(e, sdst, sdst, ep, _zeros_buf())
    return p.reshape(NPAD, H)[:N]


E_PER_TILE = E // NW
NFULL = E_PER_TILE // CHUNK
TAIL = E_PER_TILE - NFULL * CHUNK


def _edge_update_body(h_hbm, e_hbm, src_hbm, dst_hbm, out_hbm,
                      sidx_v, didx_v, hs_v, hd_v, er_v,
                      sidx_t, didx_t, hs_t, hd_t, er_t, sem):
    cid = lax.axis_index("c")
    sid = lax.axis_index("s")
    tile_base = cid * (E // NC) + sid * E_PER_TILE

    def do_chunk(base, k, sidx, didx, hs, hd, er):
        pltpu.sync_copy(src_hbm.at[pl.ds(base, k)], sidx)
        pltpu.sync_copy(dst_hbm.at[pl.ds(base, k)], didx)
        pltpu.async_copy(h_hbm.at[sidx], hs, sem).wait()
        pltpu.async_copy(h_hbm.at[didx], hd, sem).wait()
        pltpu.sync_copy(e_hbm.at[pl.ds(base, k), :], er)

        def add_row(i, carry):
            r = i // 8
            c = (i % 8) * 16
            er[r, pl.ds(c, 16)] = ((er[r, pl.ds(c, 16)]
                                    + hs[r, pl.ds(c, 16)])
                                   + hd[r, pl.ds(c, 16)])
            return carry

        lax.fori_loop(0, k * 8, add_row, 0)
        pltpu.sync_copy(er, out_hbm.at[pl.ds(base, k), :])

    def loop_body(j, carry):
        do_chunk(tile_base + j * CHUNK, CHUNK, sidx_v, didx_v, hs_v, hd_v, er_v)
        return carry

    lax.fori_loop(0, NFULL, loop_body, 0)
    do_chunk(tile_base + NFULL * CHUNK, TAIL, sidx_t, didx_t, hs_t, hd_t, er_t)


def _edge_update(h, e, ssrc, sdst):
    scratch = [
        pltpu.VMEM((CHUNK,), jnp.int32),
        pltpu.VMEM((CHUNK,), jnp.int32),
        pltpu.VMEM((CHUNK, H), jnp.float32),
        pltpu.VMEM((CHUNK, H), jnp.float32),
        pltpu.VMEM((CHUNK, H), jnp.float32),
        pltpu.VMEM((TAIL,), jnp.int32),
        pltpu.VMEM((TAIL,), jnp.int32),
        pltpu.VMEM((TAIL, H), jnp.float32),
        pltpu.VMEM((TAIL, H), jnp.float32),
        pltpu.VMEM((TAIL, H), jnp.float32),
        pltpu.SemaphoreType.DMA,
    ]
    f = functools.partial(
        pl.kernel,
        _edge_update_body,
        mesh=_MESH,
        out_type=jax.ShapeDtypeStruct((E, H), jnp.float32),
        scratch_types=scratch,
    )()
    return f(h, e, ssrc, sdst)


def _permute_rows_body(ea_hbm, perm_hbm, out_hbm,
                       pidx_v, rows_v, pidx_t, rows_t, sem):
    cid = lax.axis_index("c")
    sid = lax.axis_index("s")
    tile_base = cid * (E // NC) + sid * E_PER_TILE

    def do_chunk(base, k, pidx, rows):
        pltpu.sync_copy(perm_hbm.at[pl.ds(base, k)], pidx)
        pltpu.async_copy(ea_hbm.at[pidx], rows, sem).wait()
        pltpu.sync_copy(rows, out_hbm.at[pl.ds(base, k), :])

    def loop_body(j, carry):
        do_chunk(tile_base + j * CHUNK, CHUNK, pidx_v, rows_v)
        return carry

    lax.fori_loop(0, NFULL, loop_body, 0)
    do_chunk(tile_base + NFULL * CHUNK, TAIL, pidx_t, rows_t)


def _permute_rows(table, perm):
    scratch = [
        pltpu.VMEM((CHUNK,), jnp.int32),
        pltpu.VMEM((CHUNK, H), jnp.float32),
        pltpu.VMEM((TAIL,), jnp.int32),
        pltpu.VMEM((TAIL, H), jnp.float32),
        pltpu.SemaphoreType.DMA,
    ]
    f = functools.partial(
        pl.kernel,
        _permute_rows_body,
        mesh=_MESH,
        out_type=jax.ShapeDtypeStruct((E, H), jnp.float32),
        scratch_types=scratch,
    )()
    return f(table, perm)


CHUNK_P = 80
NCHUNKS_P = N // CHUNK_P
GR_OWN = B // NW
PBUF_ROWS = 16


def _pool_body(f0_hbm, f1_hbm, f2_hbm, b_hbm, gp_hbm, zeros_hbm,
               o0_hbm, o1_hbm, o2_hbm,
               bidx_v, r0_v, r1_v, r2_v, gp_v, a0_v, a1_v, a2_v, sem):
    cid = lax.axis_index("c")
    sid = lax.axis_index("s")
    tid = cid * NS + sid
    glo = tid * GR_OWN
    ghi = glo + GR_OWN

    pltpu.sync_copy(gp_hbm, gp_v)
    nstart, nend = _extract_pair(gp_v, tid)
    cstart = nstart // CHUNK_P
    cend = (nend + (CHUNK_P - 1)) // CHUNK_P

    pltpu.sync_copy(zeros_hbm, a0_v)
    pltpu.sync_copy(zeros_hbm, a1_v)
    pltpu.sync_copy(zeros_hbm, a2_v)

    def do_chunk(j, carry):
        base = (cstart + j) * CHUNK_P
        pltpu.sync_copy(b_hbm.at[pl.ds(base, CHUNK_P)], bidx_v.at[pl.ds(0, CHUNK_P)])
        pltpu.sync_copy(f0_hbm.at[pl.ds(base, CHUNK_P), :], r0_v)
        pltpu.sync_copy(f1_hbm.at[pl.ds(base, CHUNK_P), :], r1_v)
        pltpu.sync_copy(f2_hbm.at[pl.ds(base, CHUNK_P), :], r2_v)

        def per_node(i, c2):
            lanes = lax.iota(jnp.int32, 16)
            zero16 = jnp.full((16,), 0, jnp.int32)
            w16 = bidx_v[pl.ds((i // 16) * 16, 16)]
            b_b = _bcast_elem(w16, i % 16)
            inr = (b_b >= glo) & (b_b < ghi)
            rowv = jnp.where(inr, b_b - glo, jnp.int32(8))
            for g in range(8):
                colv = lanes + (16 * g)
                fidx = rowv * H + colv
                plsc.addupdate_scatter(a0_v, [fidx], r0_v[i, pl.ds(16 * g, 16)])
                plsc.addupdate_scatter(a1_v, [fidx], r1_v[i, pl.ds(16 * g, 16)])
                plsc.addupdate_scatter(a2_v, [fidx], r2_v[i, pl.ds(16 * g, 16)])
            return c2

        lax.fori_loop(0, CHUNK_P, per_node, 0)
        return carry

    lax.fori_loop(0, cend - cstart, do_chunk, 0)

    pltpu.sync_copy(a0_v.at[pl.ds(0, 8 * H)], o0_hbm.at[pl.ds(tid * 8 * H, 8 * H)])
    pltpu.sync_copy(a1_v.at[pl.ds(0, 8 * H)], o1_hbm.at[pl.ds(tid * 8 * H, 8 * H)])
    pltpu.sync_copy(a2_v.at[pl.ds(0, 8 * H)], o2_hbm.at[pl.ds(tid * 8 * H, 8 * H)])


def _pool(h, e2n, cyc, batch, gp):
    scratch = [
        pltpu.VMEM((128,), jnp.int32),
        pltpu.VMEM((CHUNK_P, H), jnp.float32),
        pltpu.VMEM((CHUNK_P, H), jnp.float32),
        pltpu.VMEM((CHUNK_P, H), jnp.float32),
        pltpu.VMEM((48,), jnp.int32),
        pltpu.VMEM((PBUF_ROWS * H,), jnp.float32),
        pltpu.VMEM((PBUF_ROWS * H,), jnp.float32),
        pltpu.VMEM((PBUF_ROWS * H,), jnp.float32),
        pltpu.SemaphoreType.DMA,
    ]
    out_t = [jax.ShapeDtypeStruct((NW * 8 * H,), jnp.float32)] * 3
    f = functools.partial(
        pl.kernel,
        _pool_body,
        mesh=_MESH,
        out_type=out_t,
        scratch_types=scratch,
        compiler_params=pltpu.CompilerParams(needs_layout_passes=False),
    )()
    zeros = jnp.zeros((PBUF_ROWS * H,), jnp.float32)
    o0, o1, o2 = f(h, e2n, cyc, batch, gp, zeros)
    p0 = o0.reshape(NW, 8, H)[:, :GR_OWN].reshape(B, H)
    p1 = o1.reshape(NW, 8, H)[:, :GR_OWN].reshape(B, H)
    p2 = o2.reshape(NW, 8, H)[:, :GR_OWN].reshape(B, H)
    return jnp.concatenate([p0, p1, p2], axis=-1)


def kernel(x, edge_index, edge_attr, batch, W_node, W_edge, Wcm1, Wcm2, Wcm3,
           Wn1, Wn2, We1, We2, Wc1, Wc2, eps_arr, Wl1a, Wl1b, Wl2a, Wl2b, Wout):
    src = edge_index[0].astype(jnp.int32)
    dst = edge_index[1].astype(jnp.int32)
    batch = batch.astype(jnp.int32)

    perm = jnp.argsort(dst, stable=True).astype(jnp.int32)
    sdst = dst[perm]
    ssrc = src[perm]
    bounds = jnp.arange(0, NW + 1, dtype=jnp.int32) * ROWS_OWN
    ep = jnp.searchsorted(sdst, bounds, side="left").astype(jnp.int32)
    ep = jnp.pad(ep, (0, 48 - (NW + 1)))
    gbounds = jnp.arange(0, NW + 1, dtype=jnp.int32) * GR_OWN
    gp = jnp.searchsorted(batch, gbounds, side="left").astype(jnp.int32)
    gp = jnp.pad(gp, (0, 48 - (NW + 1)))

    h = x @ W_node
    e = _permute_rows(edge_attr @ W_edge, perm)

    g1 = _segsum_gather(h, ssrc, sdst, ep)
    g2 = _segsum_gather(g1, ssrc, sdst, ep)
    cyc = jnp.concatenate([g1, g2], axis=-1)
    cyc = jax.nn.relu(cyc @ Wcm1)
    cyc = jax.nn.relu(cyc @ Wcm2)
    cyc = cyc @ Wcm3
    L = Wn1.shape[0]
    for i in range(L):
        agg = _segsum_gather_plus_e(h, e, ssrc, sdst, ep)
        h_new = (1.0 + eps_arr[i]) * h + agg
        h_new = jax.nn.relu(h_new @ Wn1[i])
        h = jax.nn.relu(h_new @ Wn2[i])
        e_new = _edge_update(h, e, ssrc, sdst)
        e_new = jax.nn.relu(e_new @ We1[i])
        e = jax.nn.relu(e_new @ We2[i])
        c_new = cyc + h
        c_new = jax.nn.relu(c_new @ Wc1[i])
        cyc = jax.nn.relu(c_new @ Wc2[i])
    edge2node = _segsum_e(e, sdst, ep)
    pooled = _pool(h, edge2node, cyc, batch, gp)
    z = jax.nn.relu(pooled @ Wl1a)
    z = jax.nn.relu(z @ Wl1b)
    z = jax.nn.relu(z @ Wl2a)
    z = z @ Wl2b
    out = z @ Wout
    return out
